# Initial kernel scaffold; baseline (speedup 1.0000x reference)
import jax
import jax.numpy as jnp
from jax.experimental import pallas as pl


def kernel(batch_size, pillar_features, coords):
    raise NotImplementedError("write your pallas kernel here")


# speedup vs baseline: 1.2982x; 1.2982x over previous
import functools

import jax
import jax.numpy as jnp
from jax import lax
from jax.experimental import pallas as pl
from jax.experimental.pallas import tpu as pltpu
from jax.experimental.pallas import tpu_sc as plsc

NX = 468
NY = 468
NZ = 2
C = 64
B = 2
P = 60000

S_PLANE = NZ * NY * NX
BATCH_STRIDE = C * S_PLANE
OUT_WORDS = B * BATCH_STRIDE

NC, NS, L = 2, 16, 16
NW = NC * NS
P_PAD = 61440
PPW = P_PAD // NW
BLK = 128
NBLK = PPW // BLK
NGRP = PPW // L

_MS_ROWS = OUT_WORDS // 128
_MS_BLK = 2704


def _memset_body(o_ref):
    o_ref[...] = jnp.zeros((_MS_BLK, 128), jnp.float32)


def _zero_grid():
    return pl.pallas_call(
        _memset_body,
        out_shape=jax.ShapeDtypeStruct((_MS_ROWS, 128), jnp.float32),
        grid=(_MS_ROWS // _MS_BLK,),
        out_specs=pl.BlockSpec((_MS_BLK, 128), lambda i: (i, 0)),
    )()


_mesh = plsc.VectorSubcoreMesh(core_axis_name="c", subcore_axis_name="s")


@functools.partial(
    pl.kernel,
    out_type=(),
    mesh=_mesh,
    compiler_params=pltpu.CompilerParams(
        needs_layout_passes=False, use_tc_tiling_on_sc=False),
    scratch_types=[
        pltpu.VMEM((PPW, 4), jnp.int32),
        pltpu.VMEM((PPW,), jnp.int32),
        pltpu.VMEM((BLK // 2, 128), jnp.float32),
        pltpu.VMEM((C, BLK), jnp.float32),
        pltpu.VMEM((C, BLK), jnp.int32),
        pltpu.SemaphoreType.DMA,
        pltpu.SemaphoreType.DMA,
    ],
)
def _sc_scatter(out_ref, featp, coords, cbuf, abuf, rows, rowsT, idxb,
                gsem, ssem):
    wid = lax.axis_index("s") * NC + lax.axis_index("c")
    base = pl.multiple_of(wid * PPW, 8)

    pltpu.sync_copy(coords.at[pl.ds(base, PPW)], cbuf)

    iota = lax.iota(jnp.int32, L)
    iota_half = iota // 2
    parity64 = (iota & 1) * C

    def grp(g, _):
        r = g * L + iota
        bcol = plsc.load_gather(cbuf, [r, jnp.full((L,), 0, jnp.int32)])
        zcol = plsc.load_gather(cbuf, [r, jnp.full((L,), 1, jnp.int32)])
        ycol = plsc.load_gather(cbuf, [r, jnp.full((L,), 2, jnp.int32)])
        xcol = plsc.load_gather(cbuf, [r, jnp.full((L,), 3, jnp.int32)])
        a = bcol * BATCH_STRIDE + zcol * (NY * NX) + ycol * NX + xcol
        abuf[pl.ds(g * L, L)] = a
        return 0

    lax.fori_loop(0, NGRP, grp, 0)

    def blk(t, _):
        tb = t * BLK
        pltpu.async_copy(
            featp.at[pl.ds(pl.multiple_of((base + tb) // 2, 8), BLK // 2)],
            rows, gsem).wait()

        def chan(c, _):
            for m in range(BLK // L):
                a16 = abuf[pl.ds(tb + m * L, L)]
                idxb[c, pl.ds(m * L, L)] = a16 + c * S_PLANE
                rowsT[c, pl.ds(m * L, L)] = plsc.load_gather(
                    rows, [m * (L // 2) + iota_half, parity64 + c])
            pltpu.async_copy(rowsT.at[c], out_ref.at[idxb.at[c]], ssem)
            return 0

        lax.fori_loop(0, C, chan, 0)

        def drain(c, _):
            pltpu.make_async_copy(rowsT.at[c], out_ref.at[idxb.at[c]],
                                  ssem).wait()
            return 0

        lax.fori_loop(0, C, drain, 0)
        return 0

    lax.fori_loop(0, NBLK, blk, 0)


def kernel(batch_size, pillar_features, coords):
    del batch_size
    featp = jnp.concatenate(
        [pillar_features,
         jnp.broadcast_to(pillar_features[:1], (P_PAD - P, C))]
    ).reshape(P_PAD // 2, 2 * C)
    coordsp = jnp.concatenate(
        [coords, jnp.broadcast_to(coords[:1], (P_PAD - P, 4))])
    ref = jax.new_ref(_zero_grid().reshape(OUT_WORDS))
    _sc_scatter(ref, featp, coordsp)
    return ref[...].reshape(B, C * NZ, NY, NX)


# speedup vs baseline: 1.5063x; 1.1603x over previous
import functools

import jax
import jax.numpy as jnp
from jax import lax
from jax.experimental import pallas as pl
from jax.experimental.pallas import tpu as pltpu
from jax.experimental.pallas import tpu_sc as plsc

NX = 468
NY = 468
NZ = 2
C = 64
B = 2
P = 60000

S_PLANE = NZ * NY * NX
BATCH_STRIDE = C * S_PLANE
OUT_WORDS = B * BATCH_STRIDE

NC, NS, L = 2, 16, 16
NW = NC * NS
PPW = P // NW
NMB = 3
MB = PPW // NMB
MBW = MB * C
ROW = 8000
NROW = MBW // ROW
JPR = ROW // C
NGRP = 118
CWIN = PPW * 4 + 4

_MS_BLK = 331776


def _memset_body(o_ref):
    o_ref[...] = jnp.zeros((_MS_BLK,), jnp.float32)


def _zero_grid():
    return pl.pallas_call(
        _memset_body,
        out_shape=jax.ShapeDtypeStruct((OUT_WORDS,), jnp.float32),
        grid=(OUT_WORDS // _MS_BLK,),
        out_specs=pl.BlockSpec((_MS_BLK,), lambda i: (i,)),
    )()


_mesh = plsc.VectorSubcoreMesh(core_axis_name="c", subcore_axis_name="s")


@functools.partial(
    pl.kernel,
    out_type=(),
    mesh=_mesh,
    compiler_params=pltpu.CompilerParams(
        needs_layout_passes=False, use_tc_tiling_on_sc=False),
    scratch_types=[
        pltpu.VMEM((CWIN,), jnp.int32),
        pltpu.VMEM((PPW + 13,), jnp.int32),
        pltpu.VMEM((MBW,), jnp.float32),
        pltpu.VMEM((NROW, ROW), jnp.int32),
        pltpu.SemaphoreType.DMA,
        pltpu.SemaphoreType.DMA,
    ],
)
def _sc_scatter(out_ref, feat, coords, cbuf, abuf, rows, idxb, gsem, ssem):
    wid = lax.axis_index("s") * NC + lax.axis_index("c")
    base = wid * PPW
    cw0 = pl.multiple_of((base * 4 // 8) * 8, 8)
    off = base * 4 - cw0

    pltpu.sync_copy(coords.at[pl.ds(cw0, CWIN)], cbuf)

    iota = lax.iota(jnp.int32, L)
    cseg = [(iota + 16 * q) * S_PLANE for q in range(4)]

    def grp(g, _):
        j4 = jnp.minimum(g * L + iota, PPW - 1) * 4 + off
        bcol = plsc.load_gather(cbuf, [j4])
        zcol = plsc.load_gather(cbuf, [j4 + 1])
        ycol = plsc.load_gather(cbuf, [j4 + 2])
        xcol = plsc.load_gather(cbuf, [j4 + 3])
        a = bcol * BATCH_STRIDE + zcol * (NY * NX) + ycol * NX + xcol
        abuf[pl.ds(g * L, L)] = a
        return 0

    lax.fori_loop(0, NGRP, grp, 0)

    def mblk(mb, _):
        jbase = mb * MB
        pltpu.async_copy(
            feat.at[pl.ds(pl.multiple_of((base + jbase) * C, 8), MBW)],
            rows, gsem).wait()

        def row(d, _):
            def quad(k, _):
                j = d * JPR + k
                aj = plsc.load_gather(abuf, [jnp.full((L,), jbase + j,
                                                      jnp.int32)])
                for q in range(4):
                    idxb[d, pl.ds(k * C + q * L, L)] = aj + cseg[q]
                return 0

            lax.fori_loop(0, JPR, quad, 0)
            pltpu.async_copy(rows.at[pl.ds(d * ROW, ROW)],
                             out_ref.at[idxb.at[d]], ssem)
            return 0

        lax.fori_loop(0, NROW, row, 0)

        def drain(d, _):
            pltpu.make_async_copy(rows.at[pl.ds(d * ROW, ROW)],
                                  out_ref.at[idxb.at[d]], ssem).wait()
            return 0

        lax.fori_loop(0, NROW, drain, 0)
        return 0

    lax.fori_loop(0, NMB, mblk, 0)


def kernel(batch_size, pillar_features, coords):
    del batch_size
    ref = jax.new_ref(_zero_grid())
    _sc_scatter(ref, pillar_features.reshape(P * C), coords.reshape(P * 4))
    return ref[...].reshape(B, C * NZ, NY, NX)


# speedup vs baseline: 5.3994x; 3.5844x over previous
import functools

import jax
import jax.numpy as jnp
from jax import lax
from jax.experimental import pallas as pl
from jax.experimental.pallas import tpu as pltpu
from jax.experimental.pallas import tpu_sc as plsc

NX = 468
NY = 468
NZ = 2
C = 64
B = 2
P = 60000

S_PLANE = NZ * NY * NX
BATCH_STRIDE = C * S_PLANE
OUT_WORDS = B * BATCH_STRIDE

NC, NS, L = 2, 16, 16
FT_COLS = 60160
CH = 3752
CH_LAST = P - 15 * CH
NGRPS = 235
NR = 32
RWORDS = B * S_PLANE
PAIR = 109512
SL_EVEN = 54760
SL_ODD = 54752
ZC1 = 36512
ZC2_EVEN = SL_EVEN - ZC1
ZC2_ODD = SL_ODD - ZC1


def _tr_body(x_ref, o_ref):
    o_ref[...] = jnp.pad(x_ref[...].T, ((0, 0), (0, FT_COLS - P)))


def _transpose_feat(feat):
    return pl.pallas_call(
        _tr_body,
        out_shape=jax.ShapeDtypeStruct((C, FT_COLS), jnp.float32),
        compiler_params=pltpu.CompilerParams(
            vmem_limit_bytes=100 * 1024 * 1024),
    )(feat)


_mesh = plsc.VectorSubcoreMesh(core_axis_name="c", subcore_axis_name="s")


@functools.partial(
    pl.kernel,
    out_type=jax.ShapeDtypeStruct((OUT_WORDS,), jnp.float32),
    mesh=_mesh,
    compiler_params=pltpu.CompilerParams(
        needs_layout_passes=False, use_tc_tiling_on_sc=False),
    scratch_types=[
        pltpu.VMEM_SHARED((RWORDS + L,), jnp.float32),
        pltpu.VMEM((CH * 4,), jnp.int32),
        pltpu.VMEM((NGRPS * L,), jnp.int32),
        pltpu.VMEM((NGRPS * L,), jnp.float32),
        pltpu.VMEM((ZC1,), jnp.float32),
        pltpu.SemaphoreType.DMA,
        pltpu.SemaphoreType.DMA,
        pltpu.SemaphoreType.DMA,
    ],
)
def _sc_scatter(featT, coords, out_ref, smem, cbuf, sbase, val, zbuf,
                vsem, zsem, rsem):
    half = lax.axis_index("c")
    t = lax.axis_index("s")
    last = t == NS - 1
    cstart = pl.multiple_of(t * CH, 8)
    clen = jnp.where(last, CH_LAST, CH)
    toff = pl.multiple_of((t // 2) * PAIR + (t % 2) * SL_EVEN, 8)
    tlen = jnp.where(t % 2 == 0, SL_EVEN, SL_ODD)
    iota = lax.iota(jnp.int32, L)

    def zinit(i, _):
        zbuf[pl.ds(i * L, L)] = jnp.zeros((L,), jnp.float32)
        return 0

    lax.fori_loop(0, ZC1 // L, zinit, 0)

    @pl.when(jnp.logical_not(last))
    def _():
        pltpu.sync_copy(coords.at[pl.ds(cstart * 4, CH * 4)], cbuf)

    @pl.when(last)
    def _():
        pltpu.sync_copy(coords.at[pl.ds(cstart * 4, CH_LAST * 4)],
                        cbuf.at[pl.ds(0, CH_LAST * 4)])

    def grp(g, _):
        j4 = jnp.minimum(g * L + iota, clen - 1) * 4
        bcol = plsc.load_gather(cbuf, [j4])
        zcol = plsc.load_gather(cbuf, [j4 + 1])
        ycol = plsc.load_gather(cbuf, [j4 + 2])
        xcol = plsc.load_gather(cbuf, [j4 + 3])
        sbase[pl.ds(g * L, L)] = (bcol * S_PLANE + zcol * (NY * NX)
                                  + ycol * NX + xcol)
        return 0

    lax.fori_loop(0, NGRPS, grp, 0)

    for g in range(NGRPS - 3, NGRPS):
        j = g * L + iota
        cur = sbase[pl.ds(g * L, L)]
        sbase[pl.ds(g * L, L)] = jnp.where(j < clen, cur, RWORDS + iota)

    def rnd(r, _):
        c = half * NR + r

        @pl.when(jnp.logical_not(last))
        def _():
            pltpu.async_copy(
                featT.at[pl.ds(c * FT_COLS + cstart, CH)],
                val.at[pl.ds(0, CH)], vsem)

        @pl.when(last)
        def _():
            pltpu.async_copy(
                featT.at[pl.ds(c * FT_COLS + cstart, CH_LAST)],
                val.at[pl.ds(0, CH_LAST)], vsem)

        pltpu.async_copy(zbuf, smem.at[pl.ds(toff, ZC1)], zsem)

        @pl.when(t % 2 == 0)
        def _():
            pltpu.async_copy(zbuf.at[pl.ds(0, ZC2_EVEN)],
                             smem.at[pl.ds(toff + ZC1, ZC2_EVEN)], zsem)

        @pl.when(t % 2 != 0)
        def _():
            pltpu.async_copy(zbuf.at[pl.ds(0, ZC2_ODD)],
                             smem.at[pl.ds(toff + ZC1, ZC2_ODD)], zsem)

        @pl.when(jnp.logical_not(last))
        def _():
            pltpu.make_async_copy(
                featT.at[pl.ds(c * FT_COLS + cstart, CH)],
                val.at[pl.ds(0, CH)], vsem).wait()

        @pl.when(last)
        def _():
            pltpu.make_async_copy(
                featT.at[pl.ds(c * FT_COLS + cstart, CH_LAST)],
                val.at[pl.ds(0, CH_LAST)], vsem).wait()

        pltpu.make_async_copy(zbuf, smem.at[pl.ds(toff, ZC1)], zsem).wait()

        @pl.when(t % 2 == 0)
        def _():
            pltpu.make_async_copy(
                zbuf.at[pl.ds(0, ZC2_EVEN)],
                smem.at[pl.ds(toff + ZC1, ZC2_EVEN)], zsem).wait()

        @pl.when(t % 2 != 0)
        def _():
            pltpu.make_async_copy(
                zbuf.at[pl.ds(0, ZC2_ODD)],
                smem.at[pl.ds(toff + ZC1, ZC2_ODD)], zsem).wait()

        plsc.subcore_barrier()

        pltpu.sync_copy(val, smem.at[sbase])

        plsc.subcore_barrier()

        b = t // 8
        wbase = pl.multiple_of(
            b * BATCH_STRIDE + c * S_PLANE + (toff - b * S_PLANE), 8)

        @pl.when(t % 2 == 0)
        def _():
            pltpu.async_copy(smem.at[pl.ds(toff, SL_EVEN)],
                             out_ref.at[pl.ds(wbase, SL_EVEN)], rsem).wait()

        @pl.when(t % 2 != 0)
        def _():
            pltpu.async_copy(smem.at[pl.ds(toff, SL_ODD)],
                             out_ref.at[pl.ds(wbase, SL_ODD)], rsem).wait()

        return 0

    lax.fori_loop(0, NR, rnd, 0)


def kernel(batch_size, pillar_features, coords):
    del batch_size
    featT = _transpose_feat(pillar_features)
    out = _sc_scatter(featT.reshape(C * FT_COLS), coords.reshape(P * 4))
    return out.reshape(B, C * NZ, NY, NX)


# speedup vs baseline: 5.4571x; 1.0107x over previous
import functools

import jax
import jax.numpy as jnp
from jax import lax
from jax.experimental import pallas as pl
from jax.experimental.pallas import tpu as pltpu
from jax.experimental.pallas import tpu_sc as plsc

NX = 468
NY = 468
NZ = 2
C = 64
B = 2
P = 60000

S_PLANE = NZ * NY * NX
BATCH_STRIDE = C * S_PLANE
OUT_WORDS = B * BATCH_STRIDE

NC, NS, L = 2, 16, 16
FT_COLS = 60160
CH = 3752
CH_LAST = P - 15 * CH
NGRPS = 235
NR = 32
RWORDS = B * S_PLANE
PAIR = 109512
SL_EVEN = 54760
SL_ODD = 54752
PLANE = NY * NX
PLANE_PAD = 219136
OUT_PAD_WORDS = B * C * NZ * PLANE_PAD
ZC1 = 36512
ZC2_EVEN = SL_EVEN - ZC1
ZC2_ODD = SL_ODD - ZC1


def _tr_body(x_ref, o_ref):
    o_ref[...] = jnp.pad(x_ref[...].T,
                         ((0, 0), (0, FT_COLS - P))).reshape(
                             C * FT_COLS // 128, 128)


def _transpose_feat(feat):
    return pl.pallas_call(
        _tr_body,
        out_shape=jax.ShapeDtypeStruct((C * FT_COLS // 128, 128),
                                       jnp.float32),
        compiler_params=pltpu.CompilerParams(
            vmem_limit_bytes=100 * 1024 * 1024),
    )(feat)


_mesh = plsc.VectorSubcoreMesh(core_axis_name="c", subcore_axis_name="s")


@functools.partial(
    pl.kernel,
    out_type=jax.ShapeDtypeStruct((OUT_WORDS,), jnp.float32),
    mesh=_mesh,
    compiler_params=pltpu.CompilerParams(
        needs_layout_passes=False, use_tc_tiling_on_sc=False),
    scratch_types=[
        pltpu.VMEM_SHARED((RWORDS + L,), jnp.float32),
        pltpu.VMEM((CH * 4,), jnp.int32),
        pltpu.VMEM((NGRPS * L,), jnp.int32),
        pltpu.VMEM((NGRPS * L,), jnp.float32),
        pltpu.VMEM((ZC1,), jnp.float32),
        pltpu.SemaphoreType.DMA,
        pltpu.SemaphoreType.DMA,
        pltpu.SemaphoreType.DMA,
    ],
)
def _sc_scatter(featT, coords, out_ref, smem, cbuf, sbase, val, zbuf,
                vsem, zsem, rsem):
    half = lax.axis_index("c")
    t = lax.axis_index("s")
    last = t == NS - 1
    cstart = pl.multiple_of(t * CH, 8)
    clen = jnp.where(last, CH_LAST, CH)
    toff = pl.multiple_of((t // 2) * PAIR + (t % 2) * SL_EVEN, 8)
    tlen = jnp.where(t % 2 == 0, SL_EVEN, SL_ODD)
    iota = lax.iota(jnp.int32, L)

    def zinit(i, _):
        zbuf[pl.ds(i * L, L)] = jnp.zeros((L,), jnp.float32)
        return 0

    lax.fori_loop(0, ZC1 // L, zinit, 0)

    @pl.when(jnp.logical_not(last))
    def _():
        pltpu.sync_copy(coords.at[pl.ds(cstart * 4, CH * 4)], cbuf)

    @pl.when(last)
    def _():
        pltpu.sync_copy(coords.at[pl.ds(cstart * 4, CH_LAST * 4)],
                        cbuf.at[pl.ds(0, CH_LAST * 4)])

    def grp(g, _):
        j4 = jnp.minimum(g * L + iota, clen - 1) * 4
        bcol = plsc.load_gather(cbuf, [j4])
        zcol = plsc.load_gather(cbuf, [j4 + 1])
        ycol = plsc.load_gather(cbuf, [j4 + 2])
        xcol = plsc.load_gather(cbuf, [j4 + 3])
        sbase[pl.ds(g * L, L)] = (bcol * S_PLANE + zcol * (NY * NX)
                                  + ycol * NX + xcol)
        return 0

    lax.fori_loop(0, NGRPS, grp, 0)

    for g in range(NGRPS - 3, NGRPS):
        j = g * L + iota
        cur = sbase[pl.ds(g * L, L)]
        sbase[pl.ds(g * L, L)] = jnp.where(j < clen, cur, RWORDS + iota)

    def rnd(r, _):
        c = half * NR + r

        @pl.when(jnp.logical_not(last))
        def _():
            pltpu.async_copy(
                featT.at[pl.ds(c * FT_COLS + cstart, CH)],
                val.at[pl.ds(0, CH)], vsem)

        @pl.when(last)
        def _():
            pltpu.async_copy(
                featT.at[pl.ds(c * FT_COLS + cstart, CH_LAST)],
                val.at[pl.ds(0, CH_LAST)], vsem)

        pltpu.async_copy(zbuf, smem.at[pl.ds(toff, ZC1)], zsem)

        @pl.when(t % 2 == 0)
        def _():
            pltpu.async_copy(zbuf.at[pl.ds(0, ZC2_EVEN)],
                             smem.at[pl.ds(toff + ZC1, ZC2_EVEN)], zsem)

        @pl.when(t % 2 != 0)
        def _():
            pltpu.async_copy(zbuf.at[pl.ds(0, ZC2_ODD)],
                             smem.at[pl.ds(toff + ZC1, ZC2_ODD)], zsem)

        @pl.when(jnp.logical_not(last))
        def _():
            pltpu.make_async_copy(
                featT.at[pl.ds(c * FT_COLS + cstart, CH)],
                val.at[pl.ds(0, CH)], vsem).wait()

        @pl.when(last)
        def _():
            pltpu.make_async_copy(
                featT.at[pl.ds(c * FT_COLS + cstart, CH_LAST)],
                val.at[pl.ds(0, CH_LAST)], vsem).wait()

        pltpu.make_async_copy(zbuf, smem.at[pl.ds(toff, ZC1)], zsem).wait()

        @pl.when(t % 2 == 0)
        def _():
            pltpu.make_async_copy(
                zbuf.at[pl.ds(0, ZC2_EVEN)],
                smem.at[pl.ds(toff + ZC1, ZC2_EVEN)], zsem).wait()

        @pl.when(t % 2 != 0)
        def _():
            pltpu.make_async_copy(
                zbuf.at[pl.ds(0, ZC2_ODD)],
                smem.at[pl.ds(toff + ZC1, ZC2_ODD)], zsem).wait()

        plsc.subcore_barrier()

        pltpu.sync_copy(val, smem.at[sbase])

        plsc.subcore_barrier()

        b = t // 8
        wbase = pl.multiple_of(
            b * BATCH_STRIDE + c * S_PLANE + (toff - b * S_PLANE), 8)

        @pl.when(t % 2 == 0)
        def _():
            pltpu.async_copy(smem.at[pl.ds(toff, SL_EVEN)],
                             out_ref.at[pl.ds(wbase, SL_EVEN)], rsem).wait()

        @pl.when(t % 2 != 0)
        def _():
            pltpu.async_copy(smem.at[pl.ds(toff, SL_ODD)],
                             out_ref.at[pl.ds(wbase, SL_ODD)], rsem).wait()

        return 0

    lax.fori_loop(0, NR, rnd, 0)


def kernel(batch_size, pillar_features, coords):
    del batch_size
    featT = _transpose_feat(pillar_features)
    out = _sc_scatter(featT.reshape(C * FT_COLS), coords.reshape(P * 4))
    return out.reshape(B, C * NZ, NY, NX)


# speedup vs baseline: 5.5878x; 1.0240x over previous
import functools

import jax
import jax.numpy as jnp
from jax import lax
from jax.experimental import pallas as pl
from jax.experimental.pallas import tpu as pltpu
from jax.experimental.pallas import tpu_sc as plsc

NX = 468
NY = 468
NZ = 2
C = 64
B = 2
P = 60000

S_PLANE = NZ * NY * NX
BATCH_STRIDE = C * S_PLANE
OUT_WORDS = B * BATCH_STRIDE

NC, NS, L = 2, 16, 16
FT_COLS = 60160
CH = 3752
CH_LAST = P - 15 * CH
NGRPS = 235
NR = 32
RWORDS = B * S_PLANE
PAIR = 109512
SL_EVEN = 54760
SL_ODD = 54752
ZC1 = 36512
ZC2_EVEN = SL_EVEN - ZC1
ZC2_ODD = SL_ODD - ZC1


def _tr_body(x_ref, o_ref):
    o_ref[...] = jnp.pad(x_ref[...].T,
                         ((0, 0), (0, FT_COLS - P))).reshape(
                             C * FT_COLS // 128, 128)


def _transpose_feat(feat):
    return pl.pallas_call(
        _tr_body,
        out_shape=jax.ShapeDtypeStruct((C * FT_COLS // 128, 128),
                                       jnp.float32),
        compiler_params=pltpu.CompilerParams(
            vmem_limit_bytes=100 * 1024 * 1024),
    )(feat)


_mesh = plsc.VectorSubcoreMesh(core_axis_name="c", subcore_axis_name="s")


@functools.partial(
    pl.kernel,
    out_type=jax.ShapeDtypeStruct((OUT_WORDS,), jnp.float32),
    mesh=_mesh,
    compiler_params=pltpu.CompilerParams(
        needs_layout_passes=False, use_tc_tiling_on_sc=False),
    scratch_types=[
        pltpu.VMEM_SHARED((RWORDS + L,), jnp.float32),
        pltpu.VMEM((CH * 4,), jnp.int32),
        pltpu.VMEM((NGRPS * L,), jnp.int32),
        pltpu.VMEM((NGRPS * L,), jnp.float32),
        pltpu.VMEM((ZC1,), jnp.float32),
        pltpu.SemaphoreType.DMA,
        pltpu.SemaphoreType.DMA,
        pltpu.SemaphoreType.DMA,
    ],
)
def _sc_scatter(featT, coords, out_ref, smem, cbuf, sbase, val, zbuf,
                vsem, zsem, rsem):
    half = lax.axis_index("c")
    t = lax.axis_index("s")
    last = t == NS - 1
    cstart = pl.multiple_of(t * CH, 8)
    clen = jnp.where(last, CH_LAST, CH)
    toff = pl.multiple_of((t // 2) * PAIR + (t % 2) * SL_EVEN, 8)
    tlen = jnp.where(t % 2 == 0, SL_EVEN, SL_ODD)
    iota = lax.iota(jnp.int32, L)

    def zinit(i, _):
        zbuf[pl.ds(i * L, L)] = jnp.zeros((L,), jnp.float32)
        return 0

    lax.fori_loop(0, ZC1 // L, zinit, 0)

    @pl.when(jnp.logical_not(last))
    def _():
        pltpu.sync_copy(coords.at[pl.ds(cstart * 4, CH * 4)], cbuf)

    @pl.when(last)
    def _():
        pltpu.sync_copy(coords.at[pl.ds(cstart * 4, CH_LAST * 4)],
                        cbuf.at[pl.ds(0, CH_LAST * 4)])

    def grp(g, _):
        j4 = jnp.minimum(g * L + iota, clen - 1) * 4
        bcol = plsc.load_gather(cbuf, [j4])
        zcol = plsc.load_gather(cbuf, [j4 + 1])
        ycol = plsc.load_gather(cbuf, [j4 + 2])
        xcol = plsc.load_gather(cbuf, [j4 + 3])
        sbase[pl.ds(g * L, L)] = (bcol * S_PLANE + zcol * (NY * NX)
                                  + ycol * NX + xcol)
        return 0

    lax.fori_loop(0, NGRPS, grp, 0)

    for g in range(NGRPS - 3, NGRPS):
        j = g * L + iota
        cur = sbase[pl.ds(g * L, L)]
        sbase[pl.ds(g * L, L)] = jnp.where(j < clen, cur, RWORDS + iota)

    pltpu.async_copy(zbuf, smem.at[pl.ds(toff, ZC1)], zsem)

    @pl.when(t % 2 == 0)
    def _():
        pltpu.async_copy(zbuf.at[pl.ds(0, ZC2_EVEN)],
                         smem.at[pl.ds(toff + ZC1, ZC2_EVEN)], zsem)
        pltpu.make_async_copy(zbuf.at[pl.ds(0, ZC2_EVEN)],
                              smem.at[pl.ds(toff + ZC1, ZC2_EVEN)],
                              zsem).wait()

    @pl.when(t % 2 != 0)
    def _():
        pltpu.async_copy(zbuf.at[pl.ds(0, ZC2_ODD)],
                         smem.at[pl.ds(toff + ZC1, ZC2_ODD)], zsem)
        pltpu.make_async_copy(zbuf.at[pl.ds(0, ZC2_ODD)],
                              smem.at[pl.ds(toff + ZC1, ZC2_ODD)],
                              zsem).wait()

    pltpu.make_async_copy(zbuf, smem.at[pl.ds(toff, ZC1)], zsem).wait()

    def rnd(r, _):
        c = half * NR + r

        @pl.when(jnp.logical_not(last))
        def _():
            pltpu.async_copy(
                featT.at[pl.ds(c * FT_COLS + cstart, CH)],
                val.at[pl.ds(0, CH)], vsem)

        @pl.when(last)
        def _():
            pltpu.async_copy(
                featT.at[pl.ds(c * FT_COLS + cstart, CH_LAST)],
                val.at[pl.ds(0, CH_LAST)], vsem)

        @pl.when(jnp.logical_not(last))
        def _():
            pltpu.make_async_copy(
                featT.at[pl.ds(c * FT_COLS + cstart, CH)],
                val.at[pl.ds(0, CH)], vsem).wait()

        @pl.when(last)
        def _():
            pltpu.make_async_copy(
                featT.at[pl.ds(c * FT_COLS + cstart, CH_LAST)],
                val.at[pl.ds(0, CH_LAST)], vsem).wait()

        plsc.subcore_barrier()

        pltpu.sync_copy(val, smem.at[sbase])

        plsc.subcore_barrier()

        b = t // 8
        wbase = pl.multiple_of(
            b * BATCH_STRIDE + c * S_PLANE + (toff - b * S_PLANE), 8)

        @pl.when(t % 2 == 0)
        def _():
            pltpu.async_copy(smem.at[pl.ds(toff, SL_EVEN)],
                             out_ref.at[pl.ds(wbase, SL_EVEN)], rsem).wait()

        @pl.when(t % 2 != 0)
        def _():
            pltpu.async_copy(smem.at[pl.ds(toff, SL_ODD)],
                             out_ref.at[pl.ds(wbase, SL_ODD)], rsem).wait()

        return 0

    lax.fori_loop(0, NR, rnd, 0)


def kernel(batch_size, pillar_features, coords):
    del batch_size
    featT = _transpose_feat(pillar_features)
    out = _sc_scatter(featT.reshape(C * FT_COLS), coords.reshape(P * 4))
    return out.reshape(B, C * NZ, NY, NX)
